# baseline (speedup 1.0000x reference)
import functools

import jax
import jax.numpy as jnp
from jax import lax
from jax.experimental import pallas as pl
from jax.experimental.pallas import tpu as pltpu
from jax.experimental.pallas import tpu_sc as plsc

E = 8
BLK = 2048
ROWS_PER_DMA = 32
NBUF = 4
SUB = 256
LANE = 32


def _pos_body(ids_ref, pos_ref, offs_ref):
    ids = ids_ref[...]
    ua = jax.lax.broadcasted_iota(jnp.int32, (LANE, LANE), 0)
    ub = jax.lax.broadcasted_iota(jnp.int32, (LANE, LANE), 1)
    U = (ua <= ub).astype(jnp.float32)
    la = jax.lax.broadcasted_iota(jnp.int32, (SUB, SUB), 0)
    lb = jax.lax.broadcasted_iota(jnp.int32, (SUB, SUB), 1)
    Ls = (lb < la).astype(jnp.float32)
    kio = jax.lax.broadcasted_iota(jnp.int32, (1, 16), 1)

    pos = jnp.zeros(ids.shape, dtype=jnp.float32)
    offs = jnp.zeros((1, 16), dtype=jnp.float32)
    total = jnp.float32(0.0)
    for e in range(E):
        ind = (ids == e)
        indf = ind.astype(jnp.float32)
        lane_c = lax.dot_general(indf, U, (((1,), (0,)), ((), ())),
                                 preferred_element_type=jnp.float32)
        prev_rows = lax.dot_general(Ls, indf, (((1,), (0,)), ((), ())),
                                    preferred_element_type=jnp.float32)
        row_pre = jnp.sum(prev_rows, axis=1, keepdims=True)
        rank = lane_c + row_pre - indf
        pos = pos + jnp.where(ind, rank + total, 0.0)
        cnt = jnp.sum(indf)
        offs = offs + jnp.where(kio > e, cnt, 0.0)
        total = total + cnt
    pos_ref[...] = pos.astype(jnp.int32)
    offs_ref[...] = offs.astype(jnp.int32)


def _compute_pos(ids2d):
    return pl.pallas_call(
        _pos_body,
        out_shape=[
            jax.ShapeDtypeStruct((SUB, LANE), jnp.int32),
            jax.ShapeDtypeStruct((1, 16), jnp.int32),
        ],
    )(ids2d)


_NC, _NS = 2, 16
_NW = _NC * _NS


def _make_permute(N, D, forward):
    rows_w = N // _NW
    n_dma = rows_w // ROWS_PER_DMA
    mesh = plsc.VectorSubcoreMesh(core_axis_name="c", subcore_axis_name="s")
    buf_types = [pltpu.VMEM((ROWS_PER_DMA, D), jnp.float32)
                 for _ in range(NBUF)]
    sem_types = [pltpu.SemaphoreType.DMA for _ in range(2 * NBUF)]

    @functools.partial(
        pl.kernel,
        mesh=mesh,
        out_type=jax.ShapeDtypeStruct((N, D), jnp.float32),
        scratch_types=[pltpu.VMEM((n_dma, ROWS_PER_DMA), jnp.int32)]
        + buf_types + sem_types,
    )
    def permute(src_hbm, pos_hbm, out_hbm, idx_v, *rest):
        bufs = rest[:NBUF]
        slds = rest[NBUF : 2 * NBUF]
        ssts = rest[2 * NBUF :]
        wid = lax.axis_index("s") * _NC + lax.axis_index("c")
        c0 = wid * n_dma
        pltpu.sync_copy(pos_hbm.at[pl.ds(c0, n_dma)], idx_v)

        def load(j):
            b = bufs[j % NBUF]
            if forward:
                return pltpu.async_copy(
                    src_hbm.at[pl.ds((c0 + j) * ROWS_PER_DMA, ROWS_PER_DMA)],
                    b, slds[j % NBUF])
            return pltpu.async_copy(src_hbm.at[idx_v.at[j]], b,
                                    slds[j % NBUF])

        def store(j):
            b = bufs[j % NBUF]
            if forward:
                return pltpu.async_copy(b, out_hbm.at[idx_v.at[j]],
                                        ssts[j % NBUF])
            return pltpu.async_copy(
                b, out_hbm.at[pl.ds((c0 + j) * ROWS_PER_DMA, ROWS_PER_DMA)],
                ssts[j % NBUF])

        pending_st = [None] * NBUF
        cur_ld = {}
        for j in range(min(NBUF - 1, n_dma)):
            cur_ld[j] = load(j)
        for j in range(n_dma):
            nxt = j + NBUF - 1
            if nxt < n_dma:
                nb = nxt % NBUF
                if pending_st[nb] is not None:
                    pending_st[nb].wait()
                    pending_st[nb] = None
                cur_ld[nxt] = load(nxt)
            cur_ld[j].wait()
            pending_st[j % NBUF] = store(j)
        for b in range(NBUF):
            if pending_st[b] is not None:
                pending_st[b].wait()

    return permute


def _matmul_body(offs_ref, xs_ref, W_ref, b_ref, o_ref):
    i = pl.program_id(0)
    r0 = i * BLK

    covs = [(offs_ref[e] <= r0) & (r0 + BLK <= offs_ref[e + 1])
            for e in range(E)]
    single = covs[0]
    for e in range(1, E):
        single = single | covs[e]

    for e in range(E):
        @pl.when(covs[e])
        def _(e=e):
            o_ref[...] = (
                lax.dot_general(
                    xs_ref[...], W_ref[e],
                    (((1,), (1,)), ((), ())),
                    preferred_element_type=jnp.float32,
                )
                + b_ref[e : e + 1, :]
            )

    @pl.when(jnp.logical_not(single))
    def _():
        o_ref[...] = jnp.zeros(o_ref.shape, dtype=jnp.float32)
        rows = r0 + jax.lax.broadcasted_iota(jnp.int32, (BLK, 1), 0)
        for e in range(E):
            lo = offs_ref[e]
            hi = offs_ref[e + 1]

            @pl.when((lo < r0 + BLK) & (hi > r0))
            def _(e=e, lo=lo, hi=hi):
                m = ((rows >= lo) & (rows < hi)).astype(jnp.float32)
                xm = xs_ref[...] * m
                o_ref[...] += (
                    lax.dot_general(
                        xm, W_ref[e],
                        (((1,), (1,)), ((), ())),
                        preferred_element_type=jnp.float32,
                    )
                    + m * b_ref[e : e + 1, :]
                )


def _ragged_matmul(xs, W, b, offs):
    N, D_IN = xs.shape
    _, D_OUT, _ = W.shape
    grid_spec = pltpu.PrefetchScalarGridSpec(
        num_scalar_prefetch=1,
        grid=(N // BLK,),
        in_specs=[
            pl.BlockSpec((BLK, D_IN), lambda i, offs: (i, 0)),
            pl.BlockSpec((E, D_OUT, D_IN), lambda i, offs: (0, 0, 0)),
            pl.BlockSpec((E, D_OUT), lambda i, offs: (0, 0)),
        ],
        out_specs=pl.BlockSpec((BLK, D_OUT), lambda i, offs: (i, 0)),
    )
    return pl.pallas_call(
        _matmul_body,
        grid_spec=grid_spec,
        out_shape=jax.ShapeDtypeStruct((N, D_OUT), jnp.float32),
    )(offs, xs, W, b)


def kernel(x, class_ids, W, b):
    N, D_IN = x.shape
    _, D_OUT, _ = W.shape
    pos2d, offs2d = _compute_pos(
        class_ids.astype(jnp.int32).reshape(SUB, LANE))
    offs = offs2d.reshape(16)

    xs = _make_permute(N, D_IN, forward=True)(x, pos2d)
    ys = _ragged_matmul(xs, W, b, offs)
    out = _make_permute(N, D_OUT, forward=False)(ys, pos2d)
    return out


# speedup vs baseline: 2.3281x; 2.3281x over previous
import functools

import jax
import jax.numpy as jnp
from jax import lax
from jax.experimental import pallas as pl
from jax.experimental.pallas import tpu as pltpu
from jax.experimental.pallas import tpu_sc as plsc

E = 8
BLK = 1024
ROWS_PER_DMA = 32
NBUF = 4
SUB = 256
LANE = 32


def _pos_body(ids_ref, pos_ref, offs_ref):
    ids = ids_ref[...]
    ua = jax.lax.broadcasted_iota(jnp.int32, (LANE, LANE), 0)
    ub = jax.lax.broadcasted_iota(jnp.int32, (LANE, LANE), 1)
    U = (ua <= ub).astype(jnp.float32)
    la = jax.lax.broadcasted_iota(jnp.int32, (SUB, SUB), 0)
    lb = jax.lax.broadcasted_iota(jnp.int32, (SUB, SUB), 1)
    Ls = (lb < la).astype(jnp.float32)
    kio = jax.lax.broadcasted_iota(jnp.int32, (1, 16), 1)

    pos = jnp.zeros(ids.shape, dtype=jnp.float32)
    offs = jnp.zeros((1, 16), dtype=jnp.float32)
    total = jnp.float32(0.0)
    for e in range(E):
        ind = (ids == e)
        indf = ind.astype(jnp.float32)
        lane_c = lax.dot_general(indf, U, (((1,), (0,)), ((), ())),
                                 preferred_element_type=jnp.float32)
        prev_rows = lax.dot_general(Ls, indf, (((1,), (0,)), ((), ())),
                                    preferred_element_type=jnp.float32)
        row_pre = jnp.sum(prev_rows, axis=1, keepdims=True)
        rank = lane_c + row_pre - indf
        pos = pos + jnp.where(ind, rank + total, 0.0)
        cnt = jnp.sum(indf)
        offs = offs + jnp.where(kio > e, cnt, 0.0)
        total = total + cnt
    pos_ref[...] = pos.astype(jnp.int32)
    offs_ref[...] = offs.astype(jnp.int32)


def _compute_pos(ids2d):
    return pl.pallas_call(
        _pos_body,
        out_shape=[
            jax.ShapeDtypeStruct((SUB, LANE), jnp.int32),
            jax.ShapeDtypeStruct((1, 16), jnp.int32),
        ],
    )(ids2d)


_NC, _NS = 2, 16
_NW = _NC * _NS


def _make_permute(N, D, forward):
    rows_w = N // _NW
    n_dma = rows_w // ROWS_PER_DMA
    mesh = plsc.VectorSubcoreMesh(core_axis_name="c", subcore_axis_name="s")
    buf_types = [pltpu.VMEM((ROWS_PER_DMA, D), jnp.float32)
                 for _ in range(NBUF)]
    sem_types = [pltpu.SemaphoreType.DMA for _ in range(2 * NBUF)]

    @functools.partial(
        pl.kernel,
        mesh=mesh,
        out_type=jax.ShapeDtypeStruct((N, D), jnp.float32),
        scratch_types=[pltpu.VMEM((n_dma, ROWS_PER_DMA), jnp.int32)]
        + buf_types + sem_types,
    )
    def permute(src_hbm, pos_hbm, out_hbm, idx_v, *rest):
        bufs = rest[:NBUF]
        slds = rest[NBUF : 2 * NBUF]
        ssts = rest[2 * NBUF :]
        wid = lax.axis_index("s") * _NC + lax.axis_index("c")
        c0 = wid * n_dma
        pltpu.sync_copy(pos_hbm.at[pl.ds(c0, n_dma)], idx_v)

        def load(j):
            b = bufs[j % NBUF]
            if forward:
                return pltpu.async_copy(
                    src_hbm.at[pl.ds((c0 + j) * ROWS_PER_DMA, ROWS_PER_DMA)],
                    b, slds[j % NBUF])
            return pltpu.async_copy(src_hbm.at[idx_v.at[j]], b,
                                    slds[j % NBUF])

        def store(j):
            b = bufs[j % NBUF]
            if forward:
                return pltpu.async_copy(b, out_hbm.at[idx_v.at[j]],
                                        ssts[j % NBUF])
            return pltpu.async_copy(
                b, out_hbm.at[pl.ds((c0 + j) * ROWS_PER_DMA, ROWS_PER_DMA)],
                ssts[j % NBUF])

        pending_st = [None] * NBUF
        cur_ld = {}
        for j in range(min(NBUF - 1, n_dma)):
            cur_ld[j] = load(j)
        for j in range(n_dma):
            nxt = j + NBUF - 1
            if nxt < n_dma:
                nb = nxt % NBUF
                if pending_st[nb] is not None:
                    pending_st[nb].wait()
                    pending_st[nb] = None
                cur_ld[nxt] = load(nxt)
            cur_ld[j].wait()
            pending_st[j % NBUF] = store(j)
        for b in range(NBUF):
            if pending_st[b] is not None:
                pending_st[b].wait()

    return permute


def _matmul_body(offs_ref, xs_ref, W_ref, b_ref, o_ref):
    i = pl.program_id(0)
    r0 = i * BLK

    covs = [(offs_ref[e] <= r0) & (r0 + BLK <= offs_ref[e + 1])
            for e in range(E)]
    single = covs[0]
    for e in range(1, E):
        single = single | covs[e]

    for e in range(E):
        @pl.when(covs[e])
        def _(e=e):
            o_ref[...] = (
                lax.dot_general(
                    xs_ref[...], W_ref[e],
                    (((1,), (1,)), ((), ())),
                    preferred_element_type=jnp.float32,
                )
                + b_ref[e : e + 1, :]
            )

    @pl.when(jnp.logical_not(single))
    def _():
        o_ref[...] = jnp.zeros(o_ref.shape, dtype=jnp.float32)
        rows = r0 + jax.lax.broadcasted_iota(jnp.int32, (BLK, 1), 0)
        for e in range(E):
            lo = offs_ref[e]
            hi = offs_ref[e + 1]

            @pl.when((lo < r0 + BLK) & (hi > r0))
            def _(e=e, lo=lo, hi=hi):
                m = ((rows >= lo) & (rows < hi)).astype(jnp.float32)
                xm = xs_ref[...] * m
                o_ref[...] += (
                    lax.dot_general(
                        xm, W_ref[e],
                        (((1,), (1,)), ((), ())),
                        preferred_element_type=jnp.float32,
                    )
                    + m * b_ref[e : e + 1, :]
                )


def _ragged_matmul(xs, W, b, offs):
    N, D_IN = xs.shape
    _, D_OUT, _ = W.shape
    grid_spec = pltpu.PrefetchScalarGridSpec(
        num_scalar_prefetch=1,
        grid=(N // BLK,),
        in_specs=[
            pl.BlockSpec((BLK, D_IN), lambda i, offs: (i, 0)),
            pl.BlockSpec((E, D_OUT, D_IN), lambda i, offs: (0, 0, 0)),
            pl.BlockSpec((E, D_OUT), lambda i, offs: (0, 0)),
        ],
        out_specs=pl.BlockSpec((BLK, D_OUT), lambda i, offs: (i, 0)),
    )
    return pl.pallas_call(
        _matmul_body,
        grid_spec=grid_spec,
        out_shape=jax.ShapeDtypeStruct((N, D_OUT), jnp.float32),
    )(offs, xs, W, b)


def kernel(x, class_ids, W, b):
    N, D_IN = x.shape
    _, D_OUT, _ = W.shape
    pos2d, offs2d = _compute_pos(
        class_ids.astype(jnp.int32).reshape(SUB, LANE))
    offs = offs2d.reshape(16)

    xs = _make_permute(N, D_IN, forward=True)(x, pos2d)
    ys = _ragged_matmul(xs, W, b, offs)
    out = _make_permute(N, D_OUT, forward=False)(ys, pos2d)
    return out


# speedup vs baseline: 2.3996x; 1.0307x over previous
import functools

import jax
import jax.numpy as jnp
from jax import lax
from jax.experimental import pallas as pl
from jax.experimental.pallas import tpu as pltpu
from jax.experimental.pallas import tpu_sc as plsc

E = 8
BLK = 1024
ROWS_PER_DMA = 32
NBUF = 4
SUB = 256
LANE = 32


def _pos_body(ids_ref, pos_ref, offs_ref):
    ids = ids_ref[...]
    ua = jax.lax.broadcasted_iota(jnp.int32, (LANE, LANE), 0)
    ub = jax.lax.broadcasted_iota(jnp.int32, (LANE, LANE), 1)
    U = (ua <= ub).astype(jnp.float32)
    la = jax.lax.broadcasted_iota(jnp.int32, (SUB, SUB), 0)
    lb = jax.lax.broadcasted_iota(jnp.int32, (SUB, SUB), 1)
    Ls = (lb < la).astype(jnp.float32)
    kio = jax.lax.broadcasted_iota(jnp.int32, (1, 16), 1)

    pos = jnp.zeros(ids.shape, dtype=jnp.float32)
    offs = jnp.zeros((1, 16), dtype=jnp.float32)
    total = jnp.float32(0.0)
    for e in range(E):
        ind = (ids == e)
        indf = ind.astype(jnp.float32)
        lane_c = lax.dot_general(indf, U, (((1,), (0,)), ((), ())),
                                 preferred_element_type=jnp.float32)
        prev_rows = lax.dot_general(Ls, indf, (((1,), (0,)), ((), ())),
                                    preferred_element_type=jnp.float32)
        row_pre = jnp.sum(prev_rows, axis=1, keepdims=True)
        rank = lane_c + row_pre - indf
        pos = pos + jnp.where(ind, rank + total, 0.0)
        cnt = jnp.sum(indf)
        offs = offs + jnp.where(kio > e, cnt, 0.0)
        total = total + cnt
    pos_ref[...] = pos.astype(jnp.int32)
    offs_ref[...] = offs.astype(jnp.int32)


def _compute_pos(ids2d):
    return pl.pallas_call(
        _pos_body,
        out_shape=[
            jax.ShapeDtypeStruct((SUB, LANE), jnp.int32),
            jax.ShapeDtypeStruct((1, 16), jnp.int32),
        ],
    )(ids2d)


_NC, _NS = 2, 16
_NW = _NC * _NS


def _make_permute(N, D, forward):
    rows_w = N // _NW
    n_dma = rows_w // ROWS_PER_DMA
    mesh = plsc.VectorSubcoreMesh(core_axis_name="c", subcore_axis_name="s")
    buf_types = [pltpu.VMEM((ROWS_PER_DMA, D), jnp.float32)
                 for _ in range(NBUF)]
    sem_types = [pltpu.SemaphoreType.DMA for _ in range(2 * NBUF)]

    @functools.partial(
        pl.kernel,
        mesh=mesh,
        out_type=jax.ShapeDtypeStruct((N, D), jnp.float32),
        scratch_types=[pltpu.VMEM((n_dma, ROWS_PER_DMA), jnp.int32)]
        + buf_types + sem_types,
    )
    def permute(src_hbm, pos_hbm, out_hbm, idx_v, *rest):
        bufs = rest[:NBUF]
        slds = rest[NBUF : 2 * NBUF]
        ssts = rest[2 * NBUF :]
        wid = lax.axis_index("s") * _NC + lax.axis_index("c")
        c0 = wid * n_dma
        pltpu.sync_copy(pos_hbm.at[pl.ds(c0, n_dma)], idx_v)

        def load(j):
            b = bufs[j % NBUF]
            if forward:
                return pltpu.async_copy(
                    src_hbm.at[pl.ds((c0 + j) * ROWS_PER_DMA, ROWS_PER_DMA)],
                    b, slds[j % NBUF])
            return pltpu.async_copy(src_hbm.at[idx_v.at[j]], b,
                                    slds[j % NBUF])

        def store(j):
            b = bufs[j % NBUF]
            if forward:
                return pltpu.async_copy(b, out_hbm.at[idx_v.at[j]],
                                        ssts[j % NBUF])
            return pltpu.async_copy(
                b, out_hbm.at[pl.ds((c0 + j) * ROWS_PER_DMA, ROWS_PER_DMA)],
                ssts[j % NBUF])

        pending_st = [None] * NBUF
        cur_ld = {}
        for j in range(min(NBUF - 1, n_dma)):
            cur_ld[j] = load(j)
        for j in range(n_dma):
            nxt = j + NBUF - 1
            if nxt < n_dma:
                nb = nxt % NBUF
                if pending_st[nb] is not None:
                    pending_st[nb].wait()
                    pending_st[nb] = None
                cur_ld[nxt] = load(nxt)
            cur_ld[j].wait()
            pending_st[j % NBUF] = store(j)
        for b in range(NBUF):
            if pending_st[b] is not None:
                pending_st[b].wait()

    return permute


def _matmul_body(offs_ref, xs_ref, W_any, b_ref, o_ref, wbuf, flags, *sems):
    i = pl.program_id(0)
    r0 = i * BLK

    @pl.when(i == 0)
    def _():
        for e in range(E):
            flags[e] = 0
            pltpu.make_async_copy(W_any.at[e], wbuf.at[e], sems[e]).start()

    def ensure_w(e):
        @pl.when(flags[e] == 0)
        def _():
            pltpu.make_async_copy(W_any.at[e], wbuf.at[e], sems[e]).wait()
            flags[e] = 1

    covs = [(offs_ref[e] <= r0) & (r0 + BLK <= offs_ref[e + 1])
            for e in range(E)]
    single = covs[0]
    for e in range(1, E):
        single = single | covs[e]

    for e in range(E):
        @pl.when(covs[e])
        def _(e=e):
            ensure_w(e)
            o_ref[...] = (
                lax.dot_general(
                    xs_ref[...], wbuf[e],
                    (((1,), (1,)), ((), ())),
                    preferred_element_type=jnp.float32,
                )
                + b_ref[e : e + 1, :]
            )

    @pl.when(jnp.logical_not(single))
    def _():
        o_ref[...] = jnp.zeros(o_ref.shape, dtype=jnp.float32)
        rows = r0 + jax.lax.broadcasted_iota(jnp.int32, (BLK, 1), 0)
        for e in range(E):
            lo = offs_ref[e]
            hi = offs_ref[e + 1]

            @pl.when((lo < r0 + BLK) & (hi > r0))
            def _(e=e, lo=lo, hi=hi):
                ensure_w(e)
                m = ((rows >= lo) & (rows < hi)).astype(jnp.float32)
                xm = xs_ref[...] * m
                o_ref[...] += (
                    lax.dot_general(
                        xm, wbuf[e],
                        (((1,), (1,)), ((), ())),
                        preferred_element_type=jnp.float32,
                    )
                    + m * b_ref[e : e + 1, :]
                )


def _ragged_matmul(xs, W, b, offs):
    N, D_IN = xs.shape
    _, D_OUT, _ = W.shape
    grid_spec = pltpu.PrefetchScalarGridSpec(
        num_scalar_prefetch=1,
        grid=(N // BLK,),
        in_specs=[
            pl.BlockSpec((BLK, D_IN), lambda i, offs: (i, 0)),
            pl.BlockSpec(memory_space=pl.ANY),
            pl.BlockSpec((E, D_OUT), lambda i, offs: (0, 0)),
        ],
        out_specs=pl.BlockSpec((BLK, D_OUT), lambda i, offs: (i, 0)),
        scratch_shapes=[
            pltpu.VMEM((E, D_OUT, D_IN), jnp.float32),
            pltpu.SMEM((E,), jnp.int32),
        ] + [pltpu.SemaphoreType.DMA for _ in range(E)],
    )
    return pl.pallas_call(
        _matmul_body,
        grid_spec=grid_spec,
        out_shape=jax.ShapeDtypeStruct((N, D_OUT), jnp.float32),
    )(offs, xs, W, b)


def kernel(x, class_ids, W, b):
    N, D_IN = x.shape
    _, D_OUT, _ = W.shape
    pos2d, offs2d = _compute_pos(
        class_ids.astype(jnp.int32).reshape(SUB, LANE))
    offs = offs2d.reshape(16)

    xs = _make_permute(N, D_IN, forward=True)(x, pos2d)
    ys = _ragged_matmul(xs, W, b, offs)
    out = _make_permute(N, D_OUT, forward=False)(ys, pos2d)
    return out
